# baseline (speedup 1.0000x reference)
import functools

import jax
import jax.numpy as jnp
from jax import lax
from jax.experimental import pallas as pl
from jax.experimental.pallas import tpu as pltpu
from jax.experimental.pallas import tpu_sc as plsc

_B = 16384
_D = 32
_NC = 2
_NS = 16
_NW = _NC * _NS
_BPW = _B // _NW
_CH = 128
_NCH = _BPW // _CH
_L = 16
_G = _BPW // _L


def _cf_body(uid_hbm, iid_hbm, uemb_hbm, iemb_hbm, out_hbm,
             uidx, iidx, urows, irows, outv, sem):
    wid = lax.axis_index("s") * _NC + lax.axis_index("c")

    pltpu.sync_copy(uid_hbm.at[wid], uidx)
    pltpu.sync_copy(iid_hbm.at[wid], iidx)

    copies = []
    for i in range(_NCH):
        copies.append(pltpu.async_copy(
            uemb_hbm.at[uidx.at[i]], urows.at[pl.ds(i * _CH, _CH)], sem))
        copies.append(pltpu.async_copy(
            iemb_hbm.at[iidx.at[i]], irows.at[pl.ds(i * _CH, _CH)], sem))
    for c in copies:
        c.wait()

    lane = lax.iota(jnp.int32, _L)

    def group(g, carry):
        r0 = g * _L
        acc = jnp.zeros((_L,), jnp.float32)
        for k in range(_L):
            j = r0 + k
            u1 = urows[j, pl.ds(0, _L)]
            u2 = urows[j, pl.ds(_L, _L)]
            v1 = irows[j, pl.ds(0, _L)]
            v2 = irows[j, pl.ds(_L, _L)]
            p = u1 * v1 + u2 * v2
            acc = jnp.where(lane == k, jnp.sum(p), acc)
        outv[pl.ds(r0, _L)] = 1.0 / (1.0 + jnp.exp(-acc))
        return carry

    lax.fori_loop(0, _G, group, 0)

    pltpu.sync_copy(outv, out_hbm.at[pl.ds(wid * _BPW, _BPW)])


_cf_call = functools.partial(
    pl.kernel,
    out_type=jax.ShapeDtypeStruct((_B,), jnp.float32),
    mesh=plsc.VectorSubcoreMesh(core_axis_name="c", subcore_axis_name="s"),
    compiler_params=pltpu.CompilerParams(needs_layout_passes=False, use_tc_tiling_on_sc=False),
    scratch_types=[
        pltpu.VMEM((_NCH, _CH), jnp.int32),
        pltpu.VMEM((_NCH, _CH), jnp.int32),
        pltpu.VMEM((_BPW, _D), jnp.float32),
        pltpu.VMEM((_BPW, _D), jnp.float32),
        pltpu.VMEM((_BPW,), jnp.float32),
        pltpu.SemaphoreType.DMA,
    ],
)(_cf_body)


def kernel(user_ids, item_ids, user_emb, item_emb):
    uid = user_ids.reshape(_NW, _NCH, _CH)
    iid = item_ids.reshape(_NW, _NCH, _CH)
    return _cf_call(uid, iid, user_emb, item_emb)
